# baseline (speedup 1.0000x reference)
import functools

import jax
import jax.numpy as jnp
from jax import lax
from jax.experimental import pallas as pl
from jax.experimental.pallas import tpu as pltpu
from jax.experimental.pallas import tpu_sc as plsc

D_MODEL = 1024
D_SAE = 16384
N_TOK = 8192
TOPK = 32

NEG_INF = float("-inf")
L = 16
N_WORKERS = 32
ROWS_PER_W = N_TOK // N_WORKERS
N_GROUPS = 64
GROUP_VREGS = D_SAE // N_GROUPS // L
CAP_ELEMS = 2048


BT_ENC = 256
BD_ENC = 2048


def _enc_body(x_ref, w_ref, b_ref, out_ref, l1_ref):
    acc = jnp.dot(x_ref[...], w_ref[...], preferred_element_type=jnp.float32)
    lat = acc + b_ref[...]
    out_ref[...] = lat
    l1_ref[...] = jnp.max(lat.reshape(BT_ENC, BD_ENC // L, L), axis=2)


def _encoder(x, W_enc, b_enc):
    grid = (D_SAE // BD_ENC, N_TOK // BT_ENC)
    return pl.pallas_call(
        _enc_body,
        grid=grid,
        in_specs=[
            pl.BlockSpec((BT_ENC, D_MODEL), lambda j, i: (i, 0)),
            pl.BlockSpec((D_MODEL, BD_ENC), lambda j, i: (0, j)),
            pl.BlockSpec((1, BD_ENC), lambda j, i: (0, j)),
        ],
        out_specs=[
            pl.BlockSpec((BT_ENC, BD_ENC), lambda j, i: (i, j)),
            pl.BlockSpec((BT_ENC, BD_ENC // L), lambda j, i: (i, j)),
        ],
        out_shape=[
            jax.ShapeDtypeStruct((N_TOK, D_SAE), jnp.float32),
            jax.ShapeDtypeStruct((N_TOK, D_SAE // L), jnp.float32),
        ],
    )(x, W_enc, b_enc.reshape(1, D_SAE))


BT_DEC = 128


def _dec_body(s_ref, w_ref, b_ref, out_ref):
    s = s_ref[...].astype(jnp.bfloat16)
    acc = jnp.dot(s, w_ref[...], preferred_element_type=jnp.float32)
    out_ref[...] = acc + b_ref[...]


def _decoder(sparse, W_dec_bf16, b_dec):
    grid = (N_TOK // BT_DEC,)
    return pl.pallas_call(
        _dec_body,
        grid=grid,
        in_specs=[
            pl.BlockSpec((BT_DEC, D_SAE), lambda i: (i, 0)),
            pl.BlockSpec((D_SAE, D_MODEL), lambda i: (0, 0)),
            pl.BlockSpec((1, D_MODEL), lambda i: (0, 0)),
        ],
        out_specs=pl.BlockSpec((BT_DEC, D_MODEL), lambda i: (i, 0)),
        out_shape=jax.ShapeDtypeStruct((N_TOK, D_MODEL), jnp.float32),
    )(sparse, W_dec_bf16, b_dec.reshape(1, D_MODEL))


def _sort_kv(v, i, desc):
    return plsc.sort_key_val(v, i, descending=desc)


def _bf_max(av, ai, bv, bi):
    c = av >= bv
    return jnp.maximum(av, bv), jnp.where(c, ai, bi)


def _bf_min(av, ai, bv, bi):
    c = av >= bv
    return jnp.minimum(av, bv), jnp.where(c, bi, ai)


def _arrange32(c0v, c0i, c1v, c1i, desc):
    ev, ei = _bf_max(c0v, c0i, c1v, c1i)
    fv, fi = _bf_min(c0v, c0i, c1v, c1i)
    if desc:
        h = _sort_kv(ev, ei, True)
        l = _sort_kv(fv, fi, True)
        return h[0], h[1], l[0], l[1]
    h = _sort_kv(fv, fi, False)
    l = _sort_kv(ev, ei, False)
    return h[0], h[1], l[0], l[1]


def _kv_tree(lo, hi, leaf_fn, desc=True):
    n = hi - lo
    if n == 2:
        av, ai = leaf_fn(lo, desc)
        bv, bi = leaf_fn(lo + 1, not desc)
        return _arrange32(*_bf_max(av, ai, bv, bi),
                          *_bf_min(av, ai, bv, bi), desc=desc)
    mid = lo + n // 2
    a = _kv_tree(lo, mid, leaf_fn, desc)
    b = _kv_tree(mid, hi, leaf_fn, not desc)
    c0 = _bf_max(a[0], a[1], b[0], b[1])
    c1 = _bf_max(a[2], a[3], b[2], b[3])
    return _arrange32(*c0, *c1, desc=desc)


def _row_topk_scatter(buf, rowbase, l1buf, l1base, out_ref, prev_i):
    lane = jnp.arange(L, dtype=jnp.int32)
    zero16 = jnp.zeros((L,), jnp.float32)

    def b_leaf(g, d):
        return _sort_kv(l1buf[pl.ds(l1base + g * L, L)], lane + g * L, d)
    _, sid_hi, _, sid_lo = _kv_tree(0, N_GROUPS, b_leaf)

    def c_leaf(j, d):
        sid = sid_hi[j] if j < L else sid_lo[j - L]
        base = sid * L
        v = buf[pl.ds(rowbase + base, L)]
        return _sort_kv(v, base + lane, d)
    tv_hi, ti_hi, tv_lo, ti_lo = _kv_tree(0, 2 * L, c_leaf)

    plsc.store_scatter(out_ref, [prev_i[pl.ds(0, L)]], zero16)
    plsc.store_scatter(out_ref, [prev_i[pl.ds(L, L)]], zero16)
    plsc.store_scatter(out_ref, [ti_hi], tv_hi)
    plsc.store_scatter(out_ref, [ti_lo], tv_lo)
    prev_i[pl.ds(0, L)] = ti_hi
    prev_i[pl.ds(L, L)] = ti_lo


L1_W = D_SAE // L


def _sc_topk_scatter(latents, l1):
    mesh = plsc.VectorSubcoreMesh(core_axis_name="c", subcore_axis_name="s")

    @functools.partial(
        pl.kernel,
        out_type=jax.ShapeDtypeStruct((N_TOK, D_SAE), jnp.float32),
        mesh=mesh,
        compiler_params=pltpu.CompilerParams(needs_layout_passes=False),
        scratch_types=[
            pltpu.VMEM((2 * D_SAE,), jnp.float32),
            pltpu.VMEM((2 * L1_W,), jnp.float32),
            pltpu.VMEM((D_SAE,), jnp.float32),
            pltpu.VMEM((2 * L,), jnp.int32),
            pltpu.SemaphoreType.DMA,
            pltpu.SemaphoreType.DMA,
            pltpu.SemaphoreType.DMA,
        ],
    )
    def sc_kernel(lat_hbm, l1_hbm, out_hbm, inbuf, l1b, outbuf, prev_i,
                  sem0, sem1, osem):
        wid = lax.axis_index("s") * 2 + lax.axis_index("c")
        row0 = wid * ROWS_PER_W
        lane = jnp.arange(L, dtype=jnp.int32)

        def z_body(i, _):
            outbuf[pl.ds(i * L, L)] = jnp.zeros((L,), jnp.float32)
            return 0
        lax.fori_loop(0, D_SAE // L, z_body, 0)
        prev_i[pl.ds(0, L)] = lane
        prev_i[pl.ds(L, L)] = lane + L

        half0 = inbuf.at[pl.ds(0, D_SAE)]
        half1 = inbuf.at[pl.ds(D_SAE, D_SAE)]
        l1h0 = l1b.at[pl.ds(0, L1_W)]
        l1h1 = l1b.at[pl.ds(L1_W, L1_W)]
        pltpu.async_copy(lat_hbm.at[row0], half0, sem0)
        pltpu.async_copy(l1_hbm.at[row0], l1h0, sem0)

        def row_body(r, _):
            par = r & 1
            @pl.when((r + 1 < ROWS_PER_W) & (par == 0))
            def _():
                pltpu.async_copy(lat_hbm.at[row0 + r + 1], half1, sem1)
                pltpu.async_copy(l1_hbm.at[row0 + r + 1], l1h1, sem1)

            @pl.when((r + 1 < ROWS_PER_W) & (par == 1))
            def _():
                pltpu.async_copy(lat_hbm.at[row0 + r + 1], half0, sem0)
                pltpu.async_copy(l1_hbm.at[row0 + r + 1], l1h0, sem0)

            @pl.when(par == 0)
            def _():
                pltpu.make_async_copy(lat_hbm.at[row0 + r], half0, sem0).wait()
                pltpu.make_async_copy(l1_hbm.at[row0 + r], l1h0, sem0).wait()

            @pl.when(par == 1)
            def _():
                pltpu.make_async_copy(lat_hbm.at[row0 + r], half1, sem1).wait()
                pltpu.make_async_copy(l1_hbm.at[row0 + r], l1h1, sem1).wait()

            @pl.when(r > 0)
            def _():
                pltpu.make_async_copy(outbuf, out_hbm.at[row0 + r - 1],
                                      osem).wait()
            _row_topk_scatter(inbuf, par * D_SAE, l1b, par * L1_W, outbuf,
                              prev_i)
            pltpu.async_copy(outbuf, out_hbm.at[row0 + r], osem)
            return 0
        lax.fori_loop(0, ROWS_PER_W, row_body, 0)
        pltpu.make_async_copy(outbuf, out_hbm.at[row0 + ROWS_PER_W - 1],
                              osem).wait()

    return sc_kernel(latents, l1)


def kernel(x, W_enc, b_enc, W_dec, b_dec):
    latents, l1 = _encoder(x, W_enc, b_enc)
    sparse_latents = _sc_topk_scatter(latents, l1)
    recon = _decoder(sparse_latents, W_dec.astype(jnp.bfloat16), b_dec)
    return (recon, sparse_latents, latents)


# speedup vs baseline: 2.2904x; 2.2904x over previous
import functools

import jax
import jax.numpy as jnp
from jax import lax
from jax.experimental import pallas as pl
from jax.experimental.pallas import tpu as pltpu
from jax.experimental.pallas import tpu_sc as plsc

D_MODEL = 1024
D_SAE = 16384
N_TOK = 8192
TOPK = 32

NEG_INF = float("-inf")
L = 16
N_WORKERS = 32
ROWS_PER_W = N_TOK // N_WORKERS
N_GROUPS = 64
GROUP_VREGS = D_SAE // N_GROUPS // L
CAP_ELEMS = 2048


BT_ENC = 256
BD_ENC = 2048


def _enc_body(x_ref, w_ref, b_ref, out_ref, l1_ref):
    acc = jnp.dot(x_ref[...], w_ref[...], preferred_element_type=jnp.float32)
    lat = acc + b_ref[...]
    out_ref[...] = lat
    l1_ref[...] = jnp.max(lat.reshape(BT_ENC, L, 128), axis=1)


def _encoder(x, W_enc, b_enc):
    grid = (D_SAE // BD_ENC, N_TOK // BT_ENC)
    return pl.pallas_call(
        _enc_body,
        grid=grid,
        in_specs=[
            pl.BlockSpec((BT_ENC, D_MODEL), lambda j, i: (i, 0)),
            pl.BlockSpec((D_MODEL, BD_ENC), lambda j, i: (0, j)),
            pl.BlockSpec((1, BD_ENC), lambda j, i: (0, j)),
        ],
        out_specs=[
            pl.BlockSpec((BT_ENC, BD_ENC), lambda j, i: (i, j)),
            pl.BlockSpec((BT_ENC, BD_ENC // L), lambda j, i: (i, j)),
        ],
        out_shape=[
            jax.ShapeDtypeStruct((N_TOK, D_SAE), jnp.float32),
            jax.ShapeDtypeStruct((N_TOK, D_SAE // L), jnp.float32),
        ],
    )(x, W_enc, b_enc.reshape(1, D_SAE))


BT_DEC = 128


def _dec_body(s_ref, w_ref, b_ref, out_ref):
    s = s_ref[...].astype(jnp.bfloat16)
    acc = jnp.dot(s, w_ref[...], preferred_element_type=jnp.float32)
    out_ref[...] = acc + b_ref[...]


def _decoder(sparse, W_dec_bf16, b_dec):
    grid = (N_TOK // BT_DEC,)
    return pl.pallas_call(
        _dec_body,
        grid=grid,
        in_specs=[
            pl.BlockSpec((BT_DEC, D_SAE), lambda i: (i, 0)),
            pl.BlockSpec((D_SAE, D_MODEL), lambda i: (0, 0)),
            pl.BlockSpec((1, D_MODEL), lambda i: (0, 0)),
        ],
        out_specs=pl.BlockSpec((BT_DEC, D_MODEL), lambda i: (i, 0)),
        out_shape=jax.ShapeDtypeStruct((N_TOK, D_MODEL), jnp.float32),
    )(sparse, W_dec_bf16, b_dec.reshape(1, D_MODEL))


def _sort_kv(v, i, desc):
    return plsc.sort_key_val(v, i, descending=desc)


def _bf_max(av, ai, bv, bi):
    c = av >= bv
    return jnp.maximum(av, bv), jnp.where(c, ai, bi)


def _bf_min(av, ai, bv, bi):
    c = av >= bv
    return jnp.minimum(av, bv), jnp.where(c, bi, ai)


def _arrange32(c0v, c0i, c1v, c1i, desc):
    ev, ei = _bf_max(c0v, c0i, c1v, c1i)
    fv, fi = _bf_min(c0v, c0i, c1v, c1i)
    if desc:
        h = _sort_kv(ev, ei, True)
        l = _sort_kv(fv, fi, True)
        return h[0], h[1], l[0], l[1]
    h = _sort_kv(fv, fi, False)
    l = _sort_kv(ev, ei, False)
    return h[0], h[1], l[0], l[1]


def _kv_tree(lo, hi, leaf_fn, desc=True):
    n = hi - lo
    if n == 2:
        av, ai = leaf_fn(lo, desc)
        bv, bi = leaf_fn(lo + 1, not desc)
        return _arrange32(*_bf_max(av, ai, bv, bi),
                          *_bf_min(av, ai, bv, bi), desc=desc)
    mid = lo + n // 2
    a = _kv_tree(lo, mid, leaf_fn, desc)
    b = _kv_tree(mid, hi, leaf_fn, not desc)
    c0 = _bf_max(a[0], a[1], b[0], b[1])
    c1 = _bf_max(a[2], a[3], b[2], b[3])
    return _arrange32(*c0, *c1, desc=desc)


def _row_topk_scatter(buf, rowbase, l1buf, l1base, out_ref, prev_i):
    lane = jnp.arange(L, dtype=jnp.int32)
    zero16 = jnp.zeros((L,), jnp.float32)

    def b_leaf(g, d):
        return _sort_kv(l1buf[pl.ds(l1base + g * L, L)], lane + g * L, d)
    _, sid_hi, _, sid_lo = _kv_tree(0, N_GROUPS, b_leaf)

    def c_leaf(j, d):
        sid = sid_hi[j] if j < L else sid_lo[j - L]
        base = (sid >> 7) * 2048 + (sid & 127)
        idx = base + 128 * lane
        v = plsc.load_gather(buf, [rowbase + idx])
        return _sort_kv(v, idx, d)
    tv_hi, ti_hi, tv_lo, ti_lo = _kv_tree(0, 2 * L, c_leaf)

    plsc.store_scatter(out_ref, [prev_i[pl.ds(0, L)]], zero16)
    plsc.store_scatter(out_ref, [prev_i[pl.ds(L, L)]], zero16)
    plsc.store_scatter(out_ref, [ti_hi], tv_hi)
    plsc.store_scatter(out_ref, [ti_lo], tv_lo)
    prev_i[pl.ds(0, L)] = ti_hi
    prev_i[pl.ds(L, L)] = ti_lo


L1_W = D_SAE // L


def _sc_topk_scatter(latents, l1):
    mesh = plsc.VectorSubcoreMesh(core_axis_name="c", subcore_axis_name="s")

    @functools.partial(
        pl.kernel,
        out_type=jax.ShapeDtypeStruct((N_TOK, D_SAE), jnp.float32),
        mesh=mesh,
        compiler_params=pltpu.CompilerParams(needs_layout_passes=False),
        scratch_types=[
            pltpu.VMEM((2 * D_SAE,), jnp.float32),
            pltpu.VMEM((2 * L1_W,), jnp.float32),
            pltpu.VMEM((D_SAE,), jnp.float32),
            pltpu.VMEM((2 * L,), jnp.int32),
            pltpu.SemaphoreType.DMA,
            pltpu.SemaphoreType.DMA,
            pltpu.SemaphoreType.DMA,
        ],
    )
    def sc_kernel(lat_hbm, l1_hbm, out_hbm, inbuf, l1b, outbuf, prev_i,
                  sem0, sem1, osem):
        wid = lax.axis_index("s") * 2 + lax.axis_index("c")
        row0 = wid * ROWS_PER_W
        lane = jnp.arange(L, dtype=jnp.int32)

        def z_body(i, _):
            outbuf[pl.ds(i * L, L)] = jnp.zeros((L,), jnp.float32)
            return 0
        lax.fori_loop(0, D_SAE // L, z_body, 0)
        prev_i[pl.ds(0, L)] = lane
        prev_i[pl.ds(L, L)] = lane + L

        half0 = inbuf.at[pl.ds(0, D_SAE)]
        half1 = inbuf.at[pl.ds(D_SAE, D_SAE)]
        l1h0 = l1b.at[pl.ds(0, L1_W)]
        l1h1 = l1b.at[pl.ds(L1_W, L1_W)]
        pltpu.async_copy(lat_hbm.at[row0], half0, sem0)
        pltpu.async_copy(l1_hbm.at[row0], l1h0, sem0)

        def row_body(r, _):
            par = r & 1
            @pl.when((r + 1 < ROWS_PER_W) & (par == 0))
            def _():
                pltpu.async_copy(lat_hbm.at[row0 + r + 1], half1, sem1)
                pltpu.async_copy(l1_hbm.at[row0 + r + 1], l1h1, sem1)

            @pl.when((r + 1 < ROWS_PER_W) & (par == 1))
            def _():
                pltpu.async_copy(lat_hbm.at[row0 + r + 1], half0, sem0)
                pltpu.async_copy(l1_hbm.at[row0 + r + 1], l1h0, sem0)

            @pl.when(par == 0)
            def _():
                pltpu.make_async_copy(lat_hbm.at[row0 + r], half0, sem0).wait()
                pltpu.make_async_copy(l1_hbm.at[row0 + r], l1h0, sem0).wait()

            @pl.when(par == 1)
            def _():
                pltpu.make_async_copy(lat_hbm.at[row0 + r], half1, sem1).wait()
                pltpu.make_async_copy(l1_hbm.at[row0 + r], l1h1, sem1).wait()

            @pl.when(r > 0)
            def _():
                pltpu.make_async_copy(outbuf, out_hbm.at[row0 + r - 1],
                                      osem).wait()
            _row_topk_scatter(inbuf, par * D_SAE, l1b, par * L1_W, outbuf,
                              prev_i)
            pltpu.async_copy(outbuf, out_hbm.at[row0 + r], osem)
            return 0
        lax.fori_loop(0, ROWS_PER_W, row_body, 0)
        pltpu.make_async_copy(outbuf, out_hbm.at[row0 + ROWS_PER_W - 1],
                              osem).wait()

    return sc_kernel(latents, l1)


def kernel(x, W_enc, b_enc, W_dec, b_dec):
    latents, l1 = _encoder(x, W_enc, b_enc)
    sparse_latents = _sc_topk_scatter(latents, l1)
    recon = _decoder(sparse_latents, W_dec.astype(jnp.bfloat16), b_dec)
    return (recon, sparse_latents, latents)
